# baseline (speedup 1.0000x reference)
import functools
import math

import jax
import jax.numpy as jnp
from jax import lax
from jax.experimental import pallas as pl
from jax.experimental.pallas import tpu as pltpu
from jax.experimental.pallas import tpu_sc as plsc

VOC = 1000000
MAX_SEQ = 200
DIM = 64
SEQ = 200
BATCH = 4096
N_TOK = SEQ * BATCH
CTILE = BATCH // 128
NC, NS, L = 2, 16, 16
NW = NC * NS
PER_W = N_TOK // NW
CHUNK = 128
NBUF = 4
STEPS = PER_W // CHUNK
EPS = 1e-5
SCALE = math.sqrt(DIM)
NJ = DIM // L


def _rsqrt(v):
    i = lax.bitcast_convert_type(v, jnp.int32)
    i = jnp.full((L,), 0x5F3759DF, jnp.int32) - (i >> 1)
    y = lax.bitcast_convert_type(i, jnp.float32)
    for _ in range(2):
        y = y * (1.5 - 0.5 * v * y * y)
    return y


_DNUMS = lax.GatherDimensionNumbers(
    offset_dims=(), collapsed_slice_dims=(0,), start_index_map=(0,))


def _hsum(v, perms):
    for p in perms:
        v = v + lax.gather(v, p[:, None], _DNUMS, (1,),
                           mode=lax.GatherScatterMode.PROMISE_IN_BOUNDS)
    return v


def _body(item_hbm, pos_hbm, idxi_hbm, idxp_hbm, w_hbm, b_hbm, out_hbm,
          idxi_v, idxp_v, rows_v, pos_v, tile_v, w_v, b_v, gsem, wsem):
    wid = lax.axis_index("s") * NC + lax.axis_index("c")
    row0 = wid * STEPS
    tok0 = wid * PER_W

    pltpu.sync_copy(w_hbm, w_v)
    pltpu.sync_copy(b_hbm, b_v)
    wv = [w_v[pl.ds(j * L, L)] for j in range(NJ)]
    bv = [b_v[pl.ds(j * L, L)] for j in range(NJ)]
    lanes = lax.iota(jnp.int32, L)
    perms = [lanes ^ k for k in (8, 4, 2, 1)]
    dts = [(jnp.full((L,), j * L, jnp.int32) + lanes) >> 3 for j in range(NJ)]
    drs = [(jnp.full((L,), j * L, jnp.int32) + lanes) & 7 for j in range(NJ)]

    def gather_pair(s, k):
        return (
            pltpu.make_async_copy(item_hbm.at[idxi_v.at[k]],
                                  rows_v.at[pl.ds(k * CHUNK, CHUNK)], gsem),
            pltpu.make_async_copy(pos_hbm.at[idxp_v.at[k]],
                                  pos_v.at[pl.ds(k * CHUNK, CHUNK)], gsem),
        )

    def issue(s, k):
        pltpu.sync_copy(idxi_hbm.at[pl.ds(row0 + s, 1)],
                        idxi_v.at[pl.ds(k, 1)])
        pltpu.sync_copy(idxp_hbm.at[pl.ds(row0 + s, 1)],
                        idxp_v.at[pl.ds(k, 1)])
        for d in gather_pair(s, k):
            d.start()

    def wb_desc(s, k):
        g = row0 + s
        return pltpu.make_async_copy(
            tile_v.at[k], out_hbm.at[g // CTILE, :, g % CTILE], wsem)

    def compute(s, k):
        base = k * CHUNK
        tile = tile_v.at[k]

        def token(t, _):
            r = base + t
            x = [rows_v[r, pl.ds(j * L, L)] * SCALE + pos_v[r, pl.ds(j * L, L)]
                 for j in range(NJ)]
            tot = _hsum((x[0] + x[1]) + (x[2] + x[3]), perms)
            mean = tot * (1.0 / DIM)
            sq = [xj * xj for xj in x]
            sumsq = _hsum((sq[0] + sq[1]) + (sq[2] + sq[3]), perms)
            var = jnp.maximum(sumsq * (1.0 / DIM) - mean * mean, 0.0)
            rstd = _rsqrt(var + EPS)
            c = mean * rstd
            tsplat = jnp.full((L,), t, jnp.int32)
            for j in range(NJ):
                n = x[j] * rstd - c
                plsc.store_scatter(tile, [dts[j], drs[j], tsplat],
                                   n * wv[j] + bv[j])
            return _

        lax.fori_loop(0, CHUNK, token, None, unroll=4)

    issue(0, 0)
    issue(1, 1)

    def outer(i, _):
        for k in range(NBUF):
            s = i * NBUF + k
            for d in gather_pair(s, k):
                d.wait()
            compute(s, k)
            wb_desc(s, k).start()
            kn = (k + 2) % NBUF

            @pl.when(s >= 2)
            def _wait_wb():
                wb_desc(s, kn).wait()

            @pl.when(s + 2 < STEPS)
            def _issue_next():
                issue(s + 2, kn)
        return _

    lax.fori_loop(0, STEPS // NBUF, outer, None)

    for k in range(2):
        wb_desc(0, k).wait()


@jax.jit
def _run(input_sequence, position_ids, item_table, pos_table, ln_weight,
         ln_bias):
    idxi = input_sequence.reshape(N_TOK // CHUNK, CHUNK)
    idxp = position_ids.reshape(N_TOK // CHUNK, CHUNK)
    mesh = plsc.VectorSubcoreMesh(core_axis_name="c", subcore_axis_name="s")
    k = pl.kernel(
        _body,
        out_type=jax.ShapeDtypeStruct((SEQ, DIM // 8, CTILE, 8, CHUNK),
                                      jnp.float32),
        mesh=mesh,
        scratch_types=[
            pltpu.VMEM((NBUF, CHUNK), jnp.int32),
            pltpu.VMEM((NBUF, CHUNK), jnp.int32),
            pltpu.VMEM((NBUF * CHUNK, DIM), jnp.float32),
            pltpu.VMEM((NBUF * CHUNK, DIM), jnp.float32),
            pltpu.VMEM((NBUF, DIM // 8, 8, CHUNK), jnp.float32),
            pltpu.VMEM((DIM,), jnp.float32),
            pltpu.VMEM((DIM,), jnp.float32),
            pltpu.SemaphoreType.DMA,
            pltpu.SemaphoreType.DMA,
        ],
        compiler_params=pltpu.CompilerParams(use_tc_tiling_on_sc=False, needs_layout_passes=False),
    )
    out5 = k(item_table, pos_table, idxi, idxp, ln_weight, ln_bias)
    return out5.transpose(0, 2, 4, 1, 3).reshape(SEQ, BATCH, DIM)


def kernel(input_sequence, position_ids, item_table, pos_table, ln_weight,
           ln_bias):
    return _run(input_sequence, position_ids, item_table, pos_table,
                ln_weight, ln_bias)


# speedup vs baseline: 1.6857x; 1.6857x over previous
import functools
import math

import jax
import jax.numpy as jnp
from jax import lax
from jax.experimental import pallas as pl
from jax.experimental.pallas import tpu as pltpu
from jax.experimental.pallas import tpu_sc as plsc

VOC = 1000000
MAX_SEQ = 200
DIM = 64
SEQ = 200
BATCH = 4096
N_TOK = SEQ * BATCH
NC, NS, L = 2, 16, 16
NW = NC * NS
CHUNK = 128
NBUF = 4
NCH = N_TOK // CHUNK
PER_W = NCH // NW
ST = SEQ // 8
CT = BATCH // CHUNK
EPS = 1e-5
SCALE = math.sqrt(DIM)
NJ = DIM // L


def _rsqrt(v):
    i = lax.bitcast_convert_type(v, jnp.int32)
    i = jnp.full((L,), 0x5F3759DF, jnp.int32) - (i >> 1)
    y = lax.bitcast_convert_type(i, jnp.float32)
    for _ in range(2):
        y = y * (1.5 - 0.5 * v * y * y)
    return y


_DNUMS = lax.GatherDimensionNumbers(
    offset_dims=(), collapsed_slice_dims=(0,), start_index_map=(0,))


def _hsum(v, perms):
    for p in perms:
        v = v + lax.gather(v, p[:, None], _DNUMS, (1,),
                           mode=lax.GatherScatterMode.PROMISE_IN_BOUNDS)
    return v


def _body(item_hbm, pos_hbm, idxi_hbm, idxp_hbm, w_hbm, b_hbm, out_hbm,
          idxi_v, idxp_v, rows_v, pos_v, w_v, b_v, gsem, wsem):
    wid = lax.axis_index("s") * NC + lax.axis_index("c")
    g0 = wid * PER_W

    pltpu.sync_copy(w_hbm, w_v)
    pltpu.sync_copy(b_hbm, b_v)
    wv = [w_v[pl.ds(j * L, L)] for j in range(NJ)]
    bv = [b_v[pl.ds(j * L, L)] for j in range(NJ)]
    lanes = lax.iota(jnp.int32, L)
    perms = [lanes ^ k for k in (8, 4, 2, 1)]

    def coords(g):
        i = g // (CT * 8)
        r = g % (CT * 8)
        return i, r // 8, r % 8

    def gather_pair(k):
        return (
            pltpu.make_async_copy(item_hbm.at[idxi_v.at[k]],
                                  rows_v.at[pl.ds(k * CHUNK, CHUNK)], gsem),
            pltpu.make_async_copy(pos_hbm.at[idxp_v.at[k]],
                                  pos_v.at[pl.ds(k * CHUNK, CHUNK)], gsem),
        )

    def issue(g, k):
        i, c, j = coords(g)
        pltpu.sync_copy(idxi_hbm.at[i, c, pl.ds(j, 1)],
                        idxi_v.at[pl.ds(k, 1)])
        pltpu.sync_copy(idxp_hbm.at[i, c, pl.ds(j, 1)],
                        idxp_v.at[pl.ds(k, 1)])
        for m in range(CHUNK // L):
            sl = idxi_v[k, pl.ds(m * L, L)]
            idxi_v[k, pl.ds(m * L, L)] = sl + sl
        for d in gather_pair(k):
            d.start()

    def wb_desc(g, k):
        i, c, j = coords(g)
        return pltpu.make_async_copy(
            rows_v.at[pl.ds(k * CHUNK, CHUNK)],
            out_hbm.at[8 * i + j, pl.ds(c * CHUNK, CHUNK), pl.ds(0, DIM)],
            wsem)

    def compute(k):
        base = k * CHUNK

        def token(t, _):
            r = base + t
            x = [rows_v[r, pl.ds(j * L, L)] * SCALE + pos_v[r, pl.ds(j * L, L)]
                 for j in range(NJ)]
            tot = _hsum((x[0] + x[1]) + (x[2] + x[3]), perms)
            mean = tot * (1.0 / DIM)
            sq = [xj * xj for xj in x]
            sumsq = _hsum((sq[0] + sq[1]) + (sq[2] + sq[3]), perms)
            var = jnp.maximum(sumsq * (1.0 / DIM) - mean * mean, 0.0)
            rstd = _rsqrt(var + EPS)
            c = mean * rstd
            for j in range(NJ):
                n = x[j] * rstd - c
                rows_v[r, pl.ds(j * L, L)] = n * wv[j] + bv[j]
            return _

        lax.fori_loop(0, CHUNK, token, None, unroll=8)

    issue(g0 + 0, 0)
    issue(g0 + 1, 1)

    def outer(it, _):
        for k in range(NBUF):
            s = it * NBUF + k
            g = g0 + s
            for d in gather_pair(k):
                d.wait()
            compute(k)
            wb_desc(g, k).start()
            kn = (k + 2) % NBUF

            @pl.when(s >= 2)
            def _wait_wb():
                wb_desc(g, kn).wait()

            @pl.when(s + 2 < PER_W)
            def _issue_next():
                issue(g + 2, kn)
        return _

    lax.fori_loop(0, PER_W // NBUF, outer, None)

    for k in range(2):
        wb_desc(g0, k).wait()


@jax.jit
def _run(input_sequence, position_ids, item_table, pos_table, ln_weight,
         ln_bias):
    idxi = input_sequence.reshape(ST, 8, CT, CHUNK).transpose(0, 2, 1, 3)
    idxp = position_ids.reshape(ST, 8, CT, CHUNK).transpose(0, 2, 1, 3)
    mesh = plsc.VectorSubcoreMesh(core_axis_name="c", subcore_axis_name="s")
    k = pl.kernel(
        _body,
        out_type=jax.ShapeDtypeStruct((SEQ, BATCH, 2 * DIM), jnp.float32),
        mesh=mesh,
        scratch_types=[
            pltpu.VMEM((NBUF, CHUNK), jnp.int32),
            pltpu.VMEM((NBUF, CHUNK), jnp.int32),
            pltpu.VMEM((NBUF * CHUNK, DIM), jnp.float32),
            pltpu.VMEM((NBUF * CHUNK, DIM), jnp.float32),
            pltpu.VMEM((DIM,), jnp.float32),
            pltpu.VMEM((DIM,), jnp.float32),
            pltpu.SemaphoreType.DMA,
            pltpu.SemaphoreType.DMA,
        ],
        compiler_params=pltpu.CompilerParams(use_tc_tiling_on_sc=False),
    )
    item2m = jnp.pad(item_table, ((0, 0), (0, DIM))).reshape(2 * VOC, DIM)
    outp = k(item2m, pos_table, idxi, idxp, ln_weight, ln_bias)
    return outp[:, :, :DIM]


def kernel(input_sequence, position_ids, item_table, pos_table, ln_weight,
           ln_bias):
    return _run(input_sequence, position_ids, item_table, pos_table,
                ln_weight, ln_bias)


# speedup vs baseline: 1.8794x; 1.1149x over previous
import functools
import math

import jax
import jax.numpy as jnp
from jax import lax
from jax.experimental import pallas as pl
from jax.experimental.pallas import tpu as pltpu
from jax.experimental.pallas import tpu_sc as plsc

VOC = 1000000
MAX_SEQ = 200
DIM = 64
SEQ = 200
BATCH = 4096
N_TOK = SEQ * BATCH
NC, NS, L = 2, 16, 16
NW = NC * NS
CHUNK = 128
NBUF = 4
NCH = N_TOK // CHUNK
PER_W = NCH // NW
BLK = 40
NBLK = PER_W // BLK
ST = SEQ // 8
CT = BATCH // CHUNK
EPS = 1e-5
SCALE = math.sqrt(DIM)
NJ = DIM // L


def _rsqrt(v):
    i = lax.bitcast_convert_type(v, jnp.int32)
    i = jnp.full((L,), 0x5F3759DF, jnp.int32) - (i >> 1)
    y = lax.bitcast_convert_type(i, jnp.float32)
    for _ in range(2):
        y = y * (1.5 - 0.5 * v * y * y)
    return y


_DNUMS = lax.GatherDimensionNumbers(
    offset_dims=(), collapsed_slice_dims=(0,), start_index_map=(0,))


def _hsum(v, perms):
    for p in perms:
        v = v + lax.gather(v, p[:, None], _DNUMS, (1,),
                           mode=lax.GatherScatterMode.PROMISE_IN_BOUNDS)
    return v


def _body(item_hbm, pos_hbm, idxi_hbm, idxp_hbm, w_hbm, b_hbm, out_hbm,
          idxi_v, idxp_v, rows_v, pos_v, w_v, b_v, gsem, wsem, isem):
    wid = lax.axis_index("s") * NC + lax.axis_index("c")
    g0 = wid * PER_W

    pltpu.sync_copy(w_hbm, w_v)
    pltpu.sync_copy(b_hbm, b_v)
    wv = [w_v[pl.ds(j * L, L)] for j in range(NJ)]
    bv = [b_v[pl.ds(j * L, L)] for j in range(NJ)]
    lanes = lax.iota(jnp.int32, L)
    perms = [lanes ^ k for k in (8, 4, 2, 1)]

    def refill_descs(b):
        bi = b % 2
        row = g0 + b * BLK
        return (
            pltpu.make_async_copy(idxi_hbm.at[pl.ds(row, BLK)],
                                  idxi_v.at[bi], isem),
            pltpu.make_async_copy(idxp_hbm.at[pl.ds(row, BLK)],
                                  idxp_v.at[bi], isem),
        )

    def gather_pair(bi, u, k):
        return (
            pltpu.make_async_copy(item_hbm.at[idxi_v.at[bi, u]],
                                  rows_v.at[pl.ds(k * CHUNK, CHUNK)], gsem),
            pltpu.make_async_copy(pos_hbm.at[idxp_v.at[bi, u]],
                                  pos_v.at[pl.ds(k * CHUNK, CHUNK)], gsem),
        )

    def wb_desc(g, k):
        i = g // (CT * 8)
        r = g % (CT * 8)
        return pltpu.make_async_copy(
            rows_v.at[pl.ds(k * CHUNK, CHUNK)],
            out_hbm.at[8 * i + r % 8, pl.ds((r // 8) * CHUNK, CHUNK),
                       pl.ds(0, DIM)],
            wsem)

    def compute(k):
        base = k * CHUNK

        def token(t, _):
            r = base + t
            x = [rows_v[r, pl.ds(j * L, L)] * SCALE + pos_v[r, pl.ds(j * L, L)]
                 for j in range(NJ)]
            tot = _hsum((x[0] + x[1]) + (x[2] + x[3]), perms)
            mean = tot * (1.0 / DIM)
            sq = [xj * xj for xj in x]
            sumsq = _hsum((sq[0] + sq[1]) + (sq[2] + sq[3]), perms)
            var = jnp.maximum(sumsq * (1.0 / DIM) - mean * mean, 0.0)
            rstd = _rsqrt(var + EPS)
            c = mean * rstd
            for j in range(NJ):
                n = x[j] * rstd - c
                rows_v[r, pl.ds(j * L, L)] = n * wv[j] + bv[j]
            return _

        lax.fori_loop(0, CHUNK, token, None, unroll=4)

    for d in refill_descs(0):
        d.start()
    for d in refill_descs(0):
        d.wait()

    for b in range(NBLK):
        bi = b % 2
        if b + 1 < NBLK:
            for d in refill_descs(b + 1):
                d.start()
        if b > 0:
            for d in refill_descs(b):
                d.wait()
        for u0 in range(2):
            for d in gather_pair(bi, u0, u0):
                d.start()

        def inner(it, _, bi=bi, b=b):
            for k in range(NBUF):
                u = it * NBUF + k
                s = b * BLK + u
                g = g0 + s
                for d in gather_pair(bi, u, k):
                    d.wait()
                compute(k)
                wb_desc(g, k).start()

                @pl.when(s >= 2)
                def _wait_wb():
                    wb_desc(g, (k + 2) % NBUF).wait()

                @pl.when(u + 2 < BLK)
                def _issue_next():
                    for d in gather_pair(bi, u + 2, (k + 2) % NBUF):
                        d.start()
            return _

        lax.fori_loop(0, BLK // NBUF, inner, None)

    for k in range(2):
        wb_desc(g0, k).wait()


@jax.jit
def _run(input_sequence, position_ids, item_table, pos_table, ln_weight,
         ln_bias):
    idxi = ((input_sequence * 2).reshape(ST, 8, CT, CHUNK)
            .transpose(0, 2, 1, 3).reshape(NCH, CHUNK))
    idxp = (position_ids.reshape(ST, 8, CT, CHUNK)
            .transpose(0, 2, 1, 3).reshape(NCH, CHUNK))
    mesh = plsc.VectorSubcoreMesh(core_axis_name="c", subcore_axis_name="s")
    k = pl.kernel(
        _body,
        out_type=jax.ShapeDtypeStruct((SEQ, BATCH, 2 * DIM), jnp.float32),
        mesh=mesh,
        scratch_types=[
            pltpu.VMEM((2, BLK, CHUNK), jnp.int32),
            pltpu.VMEM((2, BLK, CHUNK), jnp.int32),
            pltpu.VMEM((NBUF * CHUNK, DIM), jnp.float32),
            pltpu.VMEM((NBUF * CHUNK, DIM), jnp.float32),
            pltpu.VMEM((DIM,), jnp.float32),
            pltpu.VMEM((DIM,), jnp.float32),
            pltpu.SemaphoreType.DMA,
            pltpu.SemaphoreType.DMA,
            pltpu.SemaphoreType.DMA,
        ],
        compiler_params=pltpu.CompilerParams(use_tc_tiling_on_sc=False),
    )
    item2m = jnp.pad(item_table, ((0, 0), (0, DIM))).reshape(2 * VOC, DIM)
    outp = k(item2m, pos_table, idxi, idxp, ln_weight, ln_bias)
    return outp[:, :, :DIM]


def kernel(input_sequence, position_ids, item_table, pos_table, ln_weight,
           ln_bias):
    return _run(input_sequence, position_ids, item_table, pos_table,
                ln_weight, ln_bias)
